# Initial kernel scaffold; baseline (speedup 1.0000x reference)
import jax
import jax.numpy as jnp
from jax.experimental import pallas as pl


def kernel(coordinates, attention_mask, codebook, coord_scale):
    raise NotImplementedError("write your pallas kernel here")


# speedup vs baseline: 1.2444x; 1.2444x over previous
import jax
import jax.numpy as jnp
from jax.experimental import pallas as pl
from jax.experimental.pallas import tpu as pltpu

_BLK = 2048
_K = 512
_D = 3


def _vq_body(x_ref, m_ref, cbt_ref, cb_ref, scale_ref, qc_ref, idx_ref,
             loss_ref, acc_ref, cnt_ref):
    i = pl.program_id(0)
    nsteps = pl.num_programs(0)

    x = x_ref[...]
    scale = scale_ref[...]
    flat = x * scale

    fx = flat[:, 0:1]
    fy = flat[:, 1:2]
    fz = flat[:, 2:3]
    fsq = fx * fx + fy * fy + fz * fz

    cbt = cbt_ref[...]
    cx = cbt[0:1, :]
    cy = cbt[1:2, :]
    cz = cbt[2:3, :]
    cb = cb_ref[...]
    csq = jnp.sum(cb * cb, axis=1)[None, :]

    cross = jax.lax.dot_general(flat, cbt, (((1,), (0,)), ((), ())),
                                preferred_element_type=jnp.float32)
    dist = fsq - 2.0 * cross + csq

    minv = jnp.min(dist, axis=1, keepdims=True)
    iota = jax.lax.broadcasted_iota(jnp.int32, (_BLK, _K), 1)
    idx = jnp.min(jnp.where(dist == minv, iota, _K), axis=1,
                  keepdims=True)

    onehot = (iota == idx).astype(jnp.float32)
    qx = jnp.sum(onehot * cx, axis=1, keepdims=True)
    qy = jnp.sum(onehot * cy, axis=1, keepdims=True)
    qz = jnp.sum(onehot * cz, axis=1, keepdims=True)
    quant = jnp.concatenate([qx, qy, qz], axis=1)

    sabs = jnp.abs(x[:, 0:1]) + jnp.abs(x[:, 1:2]) + jnp.abs(x[:, 2:3])
    valid = (sabs > 0.0) & (m_ref[...] > 0)
    vm = valid.astype(jnp.float32)

    diff = flat - quant
    part = jnp.sum((diff * diff) * vm)

    @pl.when(i == 0)
    def _init():
        acc_ref[0, 0] = 0.0
        cnt_ref[0, 0] = 0.0

    acc_ref[0, 0] += part
    cnt_ref[0, 0] += jnp.sum(vm)

    q_st = flat + (quant - flat)
    qc_ref[...] = jnp.where(valid, q_st, flat) / scale
    idx_ref[...] = jnp.where(valid, idx, 0)

    @pl.when(i == nsteps - 1)
    def _fin():
        n = jnp.maximum(cnt_ref[0, 0], 1.0)
        val = 0.25 * (acc_ref[0, 0] / (n * float(_D)))
        loss_ref[...] = jnp.full((1, 1), val, jnp.float32)


def kernel(coordinates, attention_mask, codebook, coord_scale):
    B, S, D = coordinates.shape
    N = B * S
    x = coordinates.reshape(N, D)
    m = attention_mask.reshape(N, 1).astype(jnp.float32)
    cbt = codebook.T
    scale = coord_scale.reshape(1, D)

    grid = (N // _BLK,)
    qc, idx, loss = pl.pallas_call(
        _vq_body,
        grid=grid,
        in_specs=[
            pl.BlockSpec((_BLK, D), lambda i: (i, 0)),
            pl.BlockSpec((_BLK, 1), lambda i: (i, 0)),
            pl.BlockSpec((D, _K), lambda i: (0, 0)),
            pl.BlockSpec((_K, D), lambda i: (0, 0)),
            pl.BlockSpec((1, D), lambda i: (0, 0)),
        ],
        out_specs=[
            pl.BlockSpec((_BLK, D), lambda i: (i, 0)),
            pl.BlockSpec((_BLK, 1), lambda i: (i, 0)),
            pl.BlockSpec((1, 1), lambda i: (0, 0)),
        ],
        out_shape=[
            jax.ShapeDtypeStruct((N, D), jnp.float32),
            jax.ShapeDtypeStruct((N, 1), jnp.int32),
            jax.ShapeDtypeStruct((1, 1), jnp.float32),
        ],
        scratch_shapes=[
            pltpu.SMEM((1, 1), jnp.float32),
            pltpu.SMEM((1, 1), jnp.float32),
        ],
    )(x, m, cbt, codebook, scale)

    quantized_coords = qc.reshape(B, S, D)
    vq_loss = loss[0, 0]
    indices = idx.reshape(B, S)
    return quantized_coords, vq_loss, indices


# speedup vs baseline: 2.6090x; 2.0966x over previous
import jax
import jax.numpy as jnp
from jax.experimental import pallas as pl
from jax.experimental.pallas import tpu as pltpu

_BLK = 2048
_K = 512
_D = 3


def _vq_body(xt_ref, m_ref, cb_ref, scale_ref, qc_ref, idx_ref, loss_ref,
             acc_ref, cnt_ref):
    i = pl.program_id(0)
    nsteps = pl.num_programs(0)

    xt = xt_ref[...]
    sc = scale_ref[...]
    flat = xt * sc

    fx = flat[0:1, :]
    fy = flat[1:2, :]
    fz = flat[2:3, :]
    fsq = fx * fx + fy * fy + fz * fz

    cb = cb_ref[...]
    csq = jnp.sum(cb * cb, axis=1, keepdims=True)

    cross = jax.lax.dot_general(cb, flat, (((1,), (0,)), ((), ())),
                                preferred_element_type=jnp.float32)
    dist = fsq - 2.0 * cross + csq

    minv = jnp.min(dist, axis=0, keepdims=True)
    iota = jax.lax.broadcasted_iota(jnp.int32, (_K, _BLK), 0)
    idx = jnp.min(jnp.where(dist == minv, iota, _K), axis=0,
                  keepdims=True)

    onehot = (iota == idx).astype(jnp.float32)
    qx = jnp.sum(onehot * cb[:, 0:1], axis=0, keepdims=True)
    qy = jnp.sum(onehot * cb[:, 1:2], axis=0, keepdims=True)
    qz = jnp.sum(onehot * cb[:, 2:3], axis=0, keepdims=True)
    quant = jnp.concatenate([qx, qy, qz], axis=0)

    sabs = jnp.abs(xt[0:1, :]) + jnp.abs(xt[1:2, :]) + jnp.abs(xt[2:3, :])
    valid = (sabs > 0.0) & (m_ref[...] > 0)
    vm = valid.astype(jnp.float32)

    diff = flat - quant
    part = jnp.sum((diff * diff) * vm)

    @pl.when(i == 0)
    def _init():
        acc_ref[0, 0] = 0.0
        cnt_ref[0, 0] = 0.0

    acc_ref[0, 0] += part
    cnt_ref[0, 0] += jnp.sum(vm)

    q_st = flat + (quant - flat)
    qc_ref[...] = jnp.where(valid, q_st, flat) / sc
    idx_ref[...] = jnp.where(valid, idx, 0)

    @pl.when(i == nsteps - 1)
    def _fin():
        n = jnp.maximum(cnt_ref[0, 0], 1.0)
        val = 0.25 * (acc_ref[0, 0] / (n * float(_D)))
        loss_ref[...] = jnp.full((1, 1), val, jnp.float32)


def kernel(coordinates, attention_mask, codebook, coord_scale):
    B, S, D = coordinates.shape
    N = B * S
    xt = coordinates.reshape(N, D).T
    m = attention_mask.reshape(1, N).astype(jnp.float32)
    scale = coord_scale.reshape(D, 1)

    grid = (N // _BLK,)
    qc, idx, loss = pl.pallas_call(
        _vq_body,
        grid=grid,
        in_specs=[
            pl.BlockSpec((D, _BLK), lambda i: (0, i)),
            pl.BlockSpec((1, _BLK), lambda i: (0, i)),
            pl.BlockSpec((_K, D), lambda i: (0, 0)),
            pl.BlockSpec((D, 1), lambda i: (0, 0)),
        ],
        out_specs=[
            pl.BlockSpec((D, _BLK), lambda i: (0, i)),
            pl.BlockSpec((1, _BLK), lambda i: (0, i)),
            pl.BlockSpec((1, 1), lambda i: (0, 0)),
        ],
        out_shape=[
            jax.ShapeDtypeStruct((D, N), jnp.float32),
            jax.ShapeDtypeStruct((1, N), jnp.int32),
            jax.ShapeDtypeStruct((1, 1), jnp.float32),
        ],
        scratch_shapes=[
            pltpu.SMEM((1, 1), jnp.float32),
            pltpu.SMEM((1, 1), jnp.float32),
        ],
    )(xt, m, codebook, scale)

    quantized_coords = qc.T.reshape(B, S, D)
    vq_loss = loss[0, 0]
    indices = idx.reshape(B, S)
    return quantized_coords, vq_loss, indices


# speedup vs baseline: 4.1874x; 1.6050x over previous
import jax
import jax.numpy as jnp
from jax.experimental import pallas as pl
from jax.experimental.pallas import tpu as pltpu

_BLK = 2048
_K = 512
_D = 3


def _vq_body(xt_ref, m_ref, cb_ref, cbm2_ref, cbt_ref, scale_ref, qc_ref,
             idx_ref, loss_ref, acc_ref, cnt_ref, csq_ref):
    i = pl.program_id(0)
    nsteps = pl.num_programs(0)

    xt = xt_ref[...]
    sc = scale_ref[...]
    flat = xt * sc

    fx = flat[0:1, :]
    fy = flat[1:2, :]
    fz = flat[2:3, :]
    fsq = fx * fx + fy * fy + fz * fz

    @pl.when(i == 0)
    def _csq():
        cb = cb_ref[...]
        csq_ref[...] = jnp.sum(cb * cb, axis=1, keepdims=True)

    csq = csq_ref[...]

    cross2 = jax.lax.dot_general(cbm2_ref[...], flat, (((1,), (0,)), ((), ())),
                                 preferred_element_type=jnp.float32)
    dist = fsq + cross2 + csq

    minv = jnp.min(dist, axis=0, keepdims=True)
    iota = jax.lax.broadcasted_iota(jnp.int32, (_K, _BLK), 0)
    idx = jnp.min(jnp.where(dist == minv, iota, _K), axis=0,
                  keepdims=True)

    onehot = (iota == idx).astype(jnp.float32)
    quant = jax.lax.dot_general(cbt_ref[...], onehot, (((1,), (0,)), ((), ())),
                                preferred_element_type=jnp.float32)

    sabs = jnp.abs(xt[0:1, :]) + jnp.abs(xt[1:2, :]) + jnp.abs(xt[2:3, :])
    valid = (sabs > 0.0) & (m_ref[...] > 0)
    vm = valid.astype(jnp.float32)

    diff = flat - quant
    part = jnp.sum((diff * diff) * vm)

    @pl.when(i == 0)
    def _init():
        acc_ref[0, 0] = 0.0
        cnt_ref[0, 0] = 0.0

    acc_ref[0, 0] += part
    cnt_ref[0, 0] += jnp.sum(vm)

    q_st = flat + (quant - flat)
    qc_ref[...] = jnp.where(valid, q_st, flat) / sc
    idx_ref[...] = jnp.where(valid, idx, 0)

    @pl.when(i == nsteps - 1)
    def _fin():
        n = jnp.maximum(cnt_ref[0, 0], 1.0)
        val = 0.25 * (acc_ref[0, 0] / (n * float(_D)))
        loss_ref[...] = jnp.full((1, 1), val, jnp.float32)


def kernel(coordinates, attention_mask, codebook, coord_scale):
    B, S, D = coordinates.shape
    N = B * S
    xt = coordinates.reshape(N, D).T
    m = attention_mask.reshape(1, N).astype(jnp.float32)
    scale = coord_scale.reshape(D, 1)

    grid = (N // _BLK,)
    qc, idx, loss = pl.pallas_call(
        _vq_body,
        grid=grid,
        in_specs=[
            pl.BlockSpec((D, _BLK), lambda i: (0, i)),
            pl.BlockSpec((1, _BLK), lambda i: (0, i)),
            pl.BlockSpec((_K, D), lambda i: (0, 0)),
            pl.BlockSpec((_K, D), lambda i: (0, 0)),
            pl.BlockSpec((D, _K), lambda i: (0, 0)),
            pl.BlockSpec((D, 1), lambda i: (0, 0)),
        ],
        out_specs=[
            pl.BlockSpec((D, _BLK), lambda i: (0, i)),
            pl.BlockSpec((1, _BLK), lambda i: (0, i)),
            pl.BlockSpec((1, 1), lambda i: (0, 0)),
        ],
        out_shape=[
            jax.ShapeDtypeStruct((D, N), jnp.float32),
            jax.ShapeDtypeStruct((1, N), jnp.int32),
            jax.ShapeDtypeStruct((1, 1), jnp.float32),
        ],
        scratch_shapes=[
            pltpu.SMEM((1, 1), jnp.float32),
            pltpu.SMEM((1, 1), jnp.float32),
            pltpu.VMEM((_K, 1), jnp.float32),
        ],
    )(xt, m, codebook, codebook * (-2.0), codebook.T, scale)

    quantized_coords = qc.T.reshape(B, S, D)
    vq_loss = loss[0, 0]
    indices = idx.reshape(B, S)
    return quantized_coords, vq_loss, indices


# speedup vs baseline: 4.7894x; 1.1438x over previous
import jax
import jax.numpy as jnp
from jax.experimental import pallas as pl
from jax.experimental.pallas import tpu as pltpu

_BLK = 8192
_K = 512
_D = 3


def _vq_body(xt_ref, m_ref, cb_ref, cbm2_ref, cbt_ref, scale_ref, qc_ref,
             idx_ref, loss_ref, acc_ref, cnt_ref, csq_ref):
    i = pl.program_id(0)
    nsteps = pl.num_programs(0)

    xt = xt_ref[...]
    sc = scale_ref[...]
    flat = xt * sc

    fx = flat[0:1, :]
    fy = flat[1:2, :]
    fz = flat[2:3, :]
    fsq = fx * fx + fy * fy + fz * fz

    @pl.when(i == 0)
    def _csq():
        cb = cb_ref[...]
        csq_ref[...] = jnp.sum(cb * cb, axis=1, keepdims=True)

    csq = csq_ref[...]

    cross2 = jax.lax.dot_general(cbm2_ref[...], flat, (((1,), (0,)), ((), ())),
                                 preferred_element_type=jnp.float32)
    dist = fsq + cross2 + csq

    minv = jnp.min(dist, axis=0, keepdims=True)
    iota = jax.lax.broadcasted_iota(jnp.int32, (_K, _BLK), 0)
    idx = jnp.min(jnp.where(dist == minv, iota, _K), axis=0,
                  keepdims=True)

    onehot = (iota == idx).astype(jnp.float32)
    quant = jax.lax.dot_general(cbt_ref[...], onehot, (((1,), (0,)), ((), ())),
                                preferred_element_type=jnp.float32)

    sabs = jnp.abs(xt[0:1, :]) + jnp.abs(xt[1:2, :]) + jnp.abs(xt[2:3, :])
    valid = (sabs > 0.0) & (m_ref[...] > 0)
    vm = valid.astype(jnp.float32)

    diff = flat - quant
    part = jnp.sum((diff * diff) * vm)

    @pl.when(i == 0)
    def _init():
        acc_ref[0, 0] = 0.0
        cnt_ref[0, 0] = 0.0

    acc_ref[0, 0] += part
    cnt_ref[0, 0] += jnp.sum(vm)

    q_st = flat + (quant - flat)
    qc_ref[...] = jnp.where(valid, q_st, flat) / sc
    idx_ref[...] = jnp.where(valid, idx, 0)

    @pl.when(i == nsteps - 1)
    def _fin():
        n = jnp.maximum(cnt_ref[0, 0], 1.0)
        val = 0.25 * (acc_ref[0, 0] / (n * float(_D)))
        loss_ref[...] = jnp.full((1, 1), val, jnp.float32)


def kernel(coordinates, attention_mask, codebook, coord_scale):
    B, S, D = coordinates.shape
    N = B * S
    xt = coordinates.reshape(N, D).T
    m = attention_mask.reshape(1, N).astype(jnp.float32)
    scale = coord_scale.reshape(D, 1)

    grid = (N // _BLK,)
    qc, idx, loss = pl.pallas_call(
        _vq_body,
        grid=grid,
        in_specs=[
            pl.BlockSpec((D, _BLK), lambda i: (0, i)),
            pl.BlockSpec((1, _BLK), lambda i: (0, i)),
            pl.BlockSpec((_K, D), lambda i: (0, 0)),
            pl.BlockSpec((_K, D), lambda i: (0, 0)),
            pl.BlockSpec((D, _K), lambda i: (0, 0)),
            pl.BlockSpec((D, 1), lambda i: (0, 0)),
        ],
        out_specs=[
            pl.BlockSpec((D, _BLK), lambda i: (0, i)),
            pl.BlockSpec((1, _BLK), lambda i: (0, i)),
            pl.BlockSpec((1, 1), lambda i: (0, 0)),
        ],
        out_shape=[
            jax.ShapeDtypeStruct((D, N), jnp.float32),
            jax.ShapeDtypeStruct((1, N), jnp.int32),
            jax.ShapeDtypeStruct((1, 1), jnp.float32),
        ],
        scratch_shapes=[
            pltpu.SMEM((1, 1), jnp.float32),
            pltpu.SMEM((1, 1), jnp.float32),
            pltpu.VMEM((_K, 1), jnp.float32),
        ],
    )(xt, m, codebook, codebook * (-2.0), codebook.T, scale)

    quantized_coords = qc.T.reshape(B, S, D)
    vq_loss = loss[0, 0]
    indices = idx.reshape(B, S)
    return quantized_coords, vq_loss, indices


# speedup vs baseline: 4.8970x; 1.0225x over previous
import jax
import jax.numpy as jnp
from jax.experimental import pallas as pl
from jax.experimental.pallas import tpu as pltpu

_BLK = 16384
_K = 512
_D = 3


def _vq_body(xt_ref, m_ref, cb_ref, cbm2_ref, cbt_ref, scale_ref, qc_ref,
             idx_ref, loss_ref, acc_ref, cnt_ref, csq_ref):
    i = pl.program_id(0)
    nsteps = pl.num_programs(0)

    xt = xt_ref[...]
    sc = scale_ref[...]
    flat = xt * sc

    fx = flat[0:1, :]
    fy = flat[1:2, :]
    fz = flat[2:3, :]
    fsq = fx * fx + fy * fy + fz * fz

    @pl.when(i == 0)
    def _csq():
        cb = cb_ref[...]
        csq_ref[...] = jnp.sum(cb * cb, axis=1, keepdims=True)

    csq = csq_ref[...]

    cross2 = jax.lax.dot_general(cbm2_ref[...], flat, (((1,), (0,)), ((), ())),
                                 preferred_element_type=jnp.float32)
    dist = fsq + cross2 + csq

    minv = jnp.min(dist, axis=0, keepdims=True)
    iota = jax.lax.broadcasted_iota(jnp.int32, (_K, _BLK), 0)
    idx = jnp.min(jnp.where(dist == minv, iota, _K), axis=0,
                  keepdims=True)

    onehot = (iota == idx).astype(jnp.float32)
    quant = jax.lax.dot_general(cbt_ref[...], onehot, (((1,), (0,)), ((), ())),
                                preferred_element_type=jnp.float32)

    sabs = jnp.abs(xt[0:1, :]) + jnp.abs(xt[1:2, :]) + jnp.abs(xt[2:3, :])
    valid = (sabs > 0.0) & (m_ref[...] > 0)
    vm = valid.astype(jnp.float32)

    diff = flat - quant
    part = jnp.sum((diff * diff) * vm)

    @pl.when(i == 0)
    def _init():
        acc_ref[0, 0] = 0.0
        cnt_ref[0, 0] = 0.0

    acc_ref[0, 0] += part
    cnt_ref[0, 0] += jnp.sum(vm)

    q_st = flat + (quant - flat)
    qc_ref[...] = jnp.where(valid, q_st, flat) / sc
    idx_ref[...] = jnp.where(valid, idx, 0)

    @pl.when(i == nsteps - 1)
    def _fin():
        n = jnp.maximum(cnt_ref[0, 0], 1.0)
        val = 0.25 * (acc_ref[0, 0] / (n * float(_D)))
        loss_ref[...] = jnp.full((1, 1), val, jnp.float32)


def kernel(coordinates, attention_mask, codebook, coord_scale):
    B, S, D = coordinates.shape
    N = B * S
    xt = coordinates.reshape(N, D).T
    m = attention_mask.reshape(1, N).astype(jnp.float32)
    scale = coord_scale.reshape(D, 1)

    grid = (N // _BLK,)
    qc, idx, loss = pl.pallas_call(
        _vq_body,
        grid=grid,
        in_specs=[
            pl.BlockSpec((D, _BLK), lambda i: (0, i)),
            pl.BlockSpec((1, _BLK), lambda i: (0, i)),
            pl.BlockSpec((_K, D), lambda i: (0, 0)),
            pl.BlockSpec((_K, D), lambda i: (0, 0)),
            pl.BlockSpec((D, _K), lambda i: (0, 0)),
            pl.BlockSpec((D, 1), lambda i: (0, 0)),
        ],
        out_specs=[
            pl.BlockSpec((D, _BLK), lambda i: (0, i)),
            pl.BlockSpec((1, _BLK), lambda i: (0, i)),
            pl.BlockSpec((1, 1), lambda i: (0, 0)),
        ],
        out_shape=[
            jax.ShapeDtypeStruct((D, N), jnp.float32),
            jax.ShapeDtypeStruct((1, N), jnp.int32),
            jax.ShapeDtypeStruct((1, 1), jnp.float32),
        ],
        scratch_shapes=[
            pltpu.SMEM((1, 1), jnp.float32),
            pltpu.SMEM((1, 1), jnp.float32),
            pltpu.VMEM((_K, 1), jnp.float32),
        ],
    )(xt, m, codebook, codebook * (-2.0), codebook.T, scale)

    quantized_coords = qc.T.reshape(B, S, D)
    vq_loss = loss[0, 0]
    indices = idx.reshape(B, S)
    return quantized_coords, vq_loss, indices
